# baseline (speedup 1.0000x reference)
import functools

import jax
import jax.numpy as jnp
from jax import shard_map
from jax.experimental import pallas as pl
from jax.experimental.pallas import tpu as pltpu
from jax.sharding import Mesh, NamedSharding, PartitionSpec as P

_Q_BLK = 1568
_IMG = 784
_K_BANK = 10000
_K_PAD = 10240
_CK = 2048
_N_CHUNK = _K_PAD // _CK
_D_AUG = 512
_PAD_VAL = 100.0


def _kprep_body(k_ref, kb_ref):
    i = pl.program_id(0)
    k = k_ref[...]
    row = jax.lax.broadcasted_iota(jnp.int32, k.shape, 0) + i * _CK
    k = jnp.where(row < _K_BANK, k, _PAD_VAL)
    ksq = jnp.sum(k * k, axis=1, keepdims=True)
    hi = ksq.astype(jnp.bfloat16)
    lo = (ksq - hi.astype(jnp.float32)).astype(jnp.bfloat16)
    zeros = jnp.zeros((k.shape[0], _D_AUG - k.shape[1] - 2), jnp.bfloat16)
    kb_ref[...] = jnp.concatenate(
        [k.astype(jnp.bfloat16), hi, lo, zeros], axis=1)


def _knn_body(q_ref, kb_ref, map_ref, max_ref):
    q = q_ref[...]
    qsq = jnp.sum(q * q, axis=1, keepdims=True)
    ones = jnp.ones((q.shape[0], 2), jnp.bfloat16)
    zeros = jnp.zeros((q.shape[0], _D_AUG - q.shape[1] - 2), jnp.bfloat16)
    qb = jnp.concatenate(
        [(-2.0 * q).astype(jnp.bfloat16), ones, zeros], axis=1)

    acc = jnp.full((_Q_BLK, 128), jnp.inf, jnp.float32)
    for c in range(_N_CHUNK):
        t = jax.lax.dot_general(
            qb, kb_ref[pl.ds(c * _CK, _CK), :],
            (((1,), (1,)), ((), ())),
            preferred_element_type=jnp.float32)
        for s in range(_CK // 128):
            acc = jnp.minimum(acc, t[:, s * 128:(s + 1) * 128])

    mins = jnp.min(acc, axis=1, keepdims=True)
    dist = jnp.sqrt(jnp.maximum(mins + qsq, 1e-12))
    map_ref[0, :, :] = dist
    m0 = jnp.max(dist[:_IMG], axis=(0, 1), keepdims=True)
    m1 = jnp.max(dist[_IMG:], axis=(0, 1), keepdims=True)
    max_ref[0, :, :] = jnp.concatenate([m0, m1], axis=0)


def _impl(queries, keys):
    n_blk = queries.shape[0] // _Q_BLK
    n_img = queries.shape[0] // _IMG
    d = queries.shape[1]

    kb = pl.pallas_call(
        _kprep_body,
        grid=(_N_CHUNK,),
        in_specs=[pl.BlockSpec((_CK, d), lambda i: (i, 0))],
        out_specs=pl.BlockSpec((_CK, _D_AUG), lambda i: (i, 0)),
        out_shape=jax.ShapeDtypeStruct((_K_PAD, _D_AUG), jnp.bfloat16),
    )(keys)

    amap, amax = pl.pallas_call(
        _knn_body,
        grid=(n_blk,),
        in_specs=[
            pl.BlockSpec((_Q_BLK, d), lambda i: (i, 0)),
            pl.BlockSpec((_K_PAD, _D_AUG), lambda i: (0, 0)),
        ],
        out_specs=[
            pl.BlockSpec((1, _Q_BLK, 1), lambda i: (i, 0, 0)),
            pl.BlockSpec((1, 2, 1), lambda i: (i, 0, 0)),
        ],
        out_shape=[
            jax.ShapeDtypeStruct((n_blk, _Q_BLK, 1), jnp.float32),
            jax.ShapeDtypeStruct((n_blk, 2, 1), jnp.float32),
        ],
    )(queries, kb)
    return amax.reshape(n_img), amap.reshape(n_img, 28, 28)


def _make_kernel():
    devs = jax.devices()
    n_shard = 2 if len(devs) >= 2 else 1
    mesh = Mesh(devs[:n_shard], ("x",))
    sharded = shard_map(
        _impl, mesh=mesh,
        in_specs=(P("x", None), P(None, None)),
        out_specs=(P("x"), P("x", None, None)),
        check_vma=False)
    return jax.jit(
        sharded,
        in_shardings=(NamedSharding(mesh, P("x", None)),
                      NamedSharding(mesh, P(None, None))),
        out_shardings=(NamedSharding(mesh, P("x")),
                       NamedSharding(mesh, P("x", None, None))),
    )


_kernel_impl = None


def kernel(queries, keys):
    global _kernel_impl
    if _kernel_impl is None:
        _kernel_impl = _make_kernel()
    return _kernel_impl(queries, keys)


# speedup vs baseline: 3.2701x; 3.2701x over previous
import functools

import jax
import jax.numpy as jnp
from jax.experimental import pallas as pl
from jax.experimental.pallas import tpu as pltpu

_Q_BLK = 1568
_IMG = 784
_K_BANK = 10000
_K_PAD = 10240
_CK = 2048
_N_CHUNK = _K_PAD // _CK
_PREP_CK = 2000
_D_AUG = 512
_PAD_NORM = 30000.0


def _knn_body(q_ref, k_ref, map_ref, max_ref, kb_ref):
    i = pl.program_id(0)

    @pl.when(i == 0)
    def _prep():
        d = k_ref.shape[1]
        for c in range(_K_BANK // _PREP_CK):
            k = k_ref[pl.ds(c * _PREP_CK, _PREP_CK), :]
            ksq = jnp.sum(k * k, axis=1, keepdims=True)
            hi = ksq.astype(jnp.bfloat16)
            lo = (ksq - hi.astype(jnp.float32)).astype(jnp.bfloat16)
            zeros = jnp.zeros((_PREP_CK, _D_AUG - d - 2), jnp.bfloat16)
            kb_ref[pl.ds(c * _PREP_CK, _PREP_CK), :] = jnp.concatenate(
                [k.astype(jnp.bfloat16), hi, lo, zeros], axis=1)
        n_pad = _K_PAD - _K_BANK
        col = jax.lax.broadcasted_iota(jnp.int32, (n_pad, _D_AUG), 1)
        kb_ref[pl.ds(_K_BANK, n_pad), :] = jnp.where(
            col == d, _PAD_NORM, 0.0).astype(jnp.bfloat16)

    q = q_ref[...]
    qsq = jnp.sum(q * q, axis=1, keepdims=True)
    ones = jnp.ones((q.shape[0], 2), jnp.bfloat16)
    zeros = jnp.zeros((q.shape[0], _D_AUG - q.shape[1] - 2), jnp.bfloat16)
    qb = jnp.concatenate(
        [(-2.0 * q).astype(jnp.bfloat16), ones, zeros], axis=1)

    acc = jnp.full((_Q_BLK, 128), jnp.inf, jnp.float32)
    for c in range(_N_CHUNK):
        t = jax.lax.dot_general(
            qb, kb_ref[pl.ds(c * _CK, _CK), :],
            (((1,), (1,)), ((), ())),
            preferred_element_type=jnp.float32)
        for s in range(_CK // 128):
            acc = jnp.minimum(acc, t[:, s * 128:(s + 1) * 128])

    mins = jnp.min(acc, axis=1, keepdims=True)
    dist = jnp.sqrt(jnp.maximum(mins + qsq, 1e-12))
    map_ref[0, :, :] = dist
    m0 = jnp.max(dist[:_IMG], axis=(0, 1), keepdims=True)
    m1 = jnp.max(dist[_IMG:], axis=(0, 1), keepdims=True)
    max_ref[0, :, :] = jnp.concatenate([m0, m1], axis=0)


@functools.partial(jax.jit, static_argnames=())
def kernel(queries, keys):
    n_blk = queries.shape[0] // _Q_BLK
    n_img = queries.shape[0] // _IMG
    d = queries.shape[1]

    amap, amax = pl.pallas_call(
        _knn_body,
        grid=(n_blk,),
        in_specs=[
            pl.BlockSpec((_Q_BLK, d), lambda i: (i, 0)),
            pl.BlockSpec((_K_BANK, d), lambda i: (0, 0)),
        ],
        out_specs=[
            pl.BlockSpec((1, _Q_BLK, 1), lambda i: (i, 0, 0)),
            pl.BlockSpec((1, 2, 1), lambda i: (i, 0, 0)),
        ],
        out_shape=[
            jax.ShapeDtypeStruct((n_blk, _Q_BLK, 1), jnp.float32),
            jax.ShapeDtypeStruct((n_blk, 2, 1), jnp.float32),
        ],
        scratch_shapes=[pltpu.VMEM((_K_PAD, _D_AUG), jnp.bfloat16)],
    )(queries, keys)
    return amax.reshape(n_img), amap.reshape(n_img, 28, 28)
